# Initial kernel scaffold; baseline (speedup 1.0000x reference)
import jax
import jax.numpy as jnp
from jax.experimental import pallas as pl


def kernel(nf_init, ef_init, rewards, params, edge_index, edge_type, node_type):
    raise NotImplementedError("write your pallas kernel here")


# speedup vs baseline: 6.4008x; 6.4008x over previous
import jax
import jax.numpy as jnp
from jax import lax
from jax.experimental import pallas as pl
from jax.experimental.pallas import tpu as pltpu
from jax.experimental.pallas import tpu_sc as plsc

N = 10000
E = 320000
NF = 128
EF = 16
H = 32
RH = 32
T = 5
NET = 4
NNT = 2

NW = 32
CHUNK = 128
KPW = 79
EP = NW * KPW * CHUNK
NP = 10016
AGG_PT = 632
EFC_PT = 2504
AGG_R = AGG_PT * 16
EFC_R = EFC_PT * 16


def _table_body(nf_ref, ws_ref, wd_ref, b_ref, ts_ref, td_ref):
    nf = nf_ref[...]
    ts_ref[0, 0:N, :] = jnp.dot(nf, ws_ref[0], preferred_element_type=jnp.float32)
    td_ref[0, 0:N, :] = jnp.dot(nf, wd_ref[0],
                                preferred_element_type=jnp.float32) + b_ref[0]


def _build_tables(nf, w_src, w_dst, bias):
    return pl.pallas_call(
        _table_body,
        grid=(NET,),
        in_specs=[
            pl.BlockSpec((N, NF), lambda t: (0, 0)),
            pl.BlockSpec((1, NF, 2 * H), lambda t: (t, 0, 0)),
            pl.BlockSpec((1, NF, 2 * H), lambda t: (t, 0, 0)),
            pl.BlockSpec((1, 1, 2 * H), lambda t: (t, 0, 0)),
        ],
        out_specs=[
            pl.BlockSpec((1, NP, 2 * H), lambda t: (t, 0, 0)),
            pl.BlockSpec((1, NP, 2 * H), lambda t: (t, 0, 0)),
        ],
        out_shape=[
            jax.ShapeDtypeStruct((NET, NP, 2 * H), jnp.float32),
            jax.ShapeDtypeStruct((NET, NP, 2 * H), jnp.float32),
        ],
    )(nf, w_src, w_dst, bias)


def _sc_body(table_s, table_d, gidx3, dst3, fidx3, efp, zagg, zefc,
             aggout, efcout,
             agg_s, efc_s, idxg, idxd, idxf, rows, rows2, efv, sem, sem2):
    cid = lax.axis_index("c")
    sid = lax.axis_index("s")
    wid = cid * 16 + sid
    pltpu.sync_copy(zagg, agg_s.at[pl.ds(sid * AGG_PT, AGG_PT)])
    pltpu.sync_copy(zefc, efc_s.at[pl.ds(sid * EFC_PT, EFC_PT)])
    pltpu.sync_copy(gidx3.at[wid], idxg)
    pltpu.sync_copy(dst3.at[wid], idxd)
    pltpu.sync_copy(fidx3.at[wid], idxf)
    plsc.subcore_barrier()

    def step(k, carry):
        eoff = (wid * KPW + k) * CHUNK
        pltpu.sync_copy(efp.at[pl.ds(eoff, CHUNK)], efv)
        g1 = pltpu.async_copy(table_s.at[idxg.at[k]], rows, sem)
        g2 = pltpu.async_copy(table_d.at[idxf.at[k]], rows2, sem2)
        g1.wait()
        g2.wait()
        pltpu.sync_copy(rows, agg_s.at[idxd.at[k]], add=True)
        pltpu.sync_copy(rows2, agg_s.at[idxd.at[k]], add=True)
        pltpu.sync_copy(efv, efc_s.at[idxf.at[k]], add=True)
        return carry

    lax.fori_loop(0, KPW, step, 0)
    plsc.subcore_barrier()
    pltpu.sync_copy(agg_s.at[pl.ds(sid * AGG_PT, AGG_PT)],
                    aggout.at[cid, pl.ds(sid * AGG_PT, AGG_PT)])
    pltpu.sync_copy(efc_s.at[pl.ds(sid * EFC_PT, EFC_PT)],
                    efcout.at[cid, pl.ds(sid * EFC_PT, EFC_PT)])


_SC_MESH = plsc.VectorSubcoreMesh(core_axis_name="c", subcore_axis_name="s",
                                  num_cores=2, num_subcores=16)

_sc_call = pl.kernel(
    _sc_body,
    out_type=(
        jax.ShapeDtypeStruct((2, AGG_R, 2 * H), jnp.float32),
        jax.ShapeDtypeStruct((2, EFC_R, EF), jnp.float32),
    ),
    mesh=_SC_MESH,
    compiler_params=pltpu.CompilerParams(use_tc_tiling_on_sc=False),
    scratch_types=[
        pltpu.VMEM_SHARED((AGG_R, 2 * H), jnp.float32),
        pltpu.VMEM_SHARED((EFC_R, EF), jnp.float32),
        pltpu.VMEM((KPW, CHUNK), jnp.int32),
        pltpu.VMEM((KPW, CHUNK), jnp.int32),
        pltpu.VMEM((KPW, CHUNK), jnp.int32),
        pltpu.VMEM((CHUNK, 2 * H), jnp.float32),
        pltpu.VMEM((CHUNK, 2 * H), jnp.float32),
        pltpu.VMEM((CHUNK, EF), jnp.float32),
        pltpu.SemaphoreType.DMA,
        pltpu.SemaphoreType.DMA,
    ],
)


BLK = 2000


def _post_body(agg2, efc2, nf_ref, nt, rew,
               We_r, We_m,
               Wn_r, Wn_m, bn_r, bn_m,
               Wih, wr, bih, Whh, bhh, wact, bact, Wc1, bc1, Wc2, bc2,
               out):
    nf = nf_ref[...]
    agg = agg2[0] + agg2[1]
    aggr = agg[:, :H]
    aggm = agg[:, H:]
    for t in range(NET):
        Ft = efc2[0, t] + efc2[1, t]
        aggr = aggr + jnp.dot(Ft, We_r[:, t * H:(t + 1) * H],
                              preferred_element_type=jnp.float32)
        aggm = aggm + jnp.dot(Ft, We_m[:, t * H:(t + 1) * H],
                              preferred_element_type=jnp.float32)

    ntv = nt[...]

    def node_update(aggx, Wn, bn):
        hcat = jnp.concatenate([aggx, nf], axis=1)
        ha = jnp.dot(hcat, Wn[...], preferred_element_type=jnp.float32)
        sel = jnp.where(ntv == 0, ha[:, :H] + bn[0:1, :], ha[:, H:] + bn[1:2, :])
        return jnp.tanh(sel)

    nfr = node_update(aggr, Wn_r, bn_r)
    nfm = node_update(aggm, Wn_m, bn_m)

    gx = jnp.dot(nfr, Wih[...], preferred_element_type=jnp.float32) + bih[...]
    h = jnp.zeros((BLK, RH), jnp.float32)
    for t in range(T):
        gi = gx + rew[0:1, t:t + 1] * wr[...]
        gh = jnp.dot(h, Whh[...], preferred_element_type=jnp.float32) + bhh[...]
        r = jax.nn.sigmoid(gi[:, :RH] + gh[:, :RH])
        z = jax.nn.sigmoid(gi[:, RH:2 * RH] + gh[:, RH:2 * RH])
        n = jnp.tanh(gi[:, 2 * RH:] + r * gh[:, 2 * RH:])
        h = (1.0 - z) * n + z * h

    sim2a = jnp.concatenate([h, nfm], axis=1)
    logits = jnp.dot(sim2a, wact[...], preferred_element_type=jnp.float32) + bact[0:1, :]
    crit = jnp.dot(jnp.maximum(jnp.dot(sim2a, Wc1[...], preferred_element_type=jnp.float32)
                               + bc1[...], 0.0),
                   Wc2[...], preferred_element_type=jnp.float32) + bc2[0:1, :]
    out[...] = jnp.concatenate([logits, crit], axis=1)


def _post_call(agg, efc, nf, nt, rew, *weights):
    return pl.pallas_call(
        _post_body,
        grid=(N // BLK,),
        in_specs=[
            pl.BlockSpec((2, BLK, 2 * H), lambda i: (0, i, 0)),
            pl.BlockSpec((2, NET, BLK, EF), lambda i: (0, 0, i, 0)),
            pl.BlockSpec((BLK, NF), lambda i: (i, 0)),
            pl.BlockSpec((BLK, 1), lambda i: (i, 0)),
            pl.BlockSpec((1, T), lambda i: (0, 0)),
        ] + [pl.BlockSpec(w.shape, lambda i, _r=len(w.shape): (0,) * _r)
             for w in weights],
        out_specs=pl.BlockSpec((BLK, 2), lambda i: (i, 0)),
        out_shape=jax.ShapeDtypeStruct((N, 2), jnp.float32),
    )(agg, efc, nf, nt, rew, *weights)


def _final_body(lc_ref, out_ref):
    logits = lc_ref[:, 0:1]
    crit = lc_ref[:, 1:2]
    m = jnp.max(logits, keepdims=True)
    ex = jnp.exp(logits - m)
    probs = ex / jnp.sum(ex, keepdims=True)
    val = jnp.sum(crit, keepdims=True) * (1.0 / N)
    out_ref[...] = jnp.concatenate([probs, val], axis=0)


def _final_call(lc):
    return pl.pallas_call(
        _final_body,
        out_shape=jax.ShapeDtypeStruct((N + 1, 1), jnp.float32),
    )(lc)


def kernel(nf_init, ef_init, rewards, params, edge_index, edge_type, node_type):
    p = params
    src = edge_index[0]
    dst = edge_index[1]
    et = edge_type
    gidx = et * NP + src
    fidx = et * NP + dst
    pad = EP - E
    gidx3 = jnp.concatenate([gidx, jnp.full((pad,), N, jnp.int32)]).reshape(NW, KPW, CHUNK)
    dst3 = jnp.concatenate([dst, jnp.full((pad,), N, jnp.int32)]).reshape(NW, KPW, CHUNK)
    fidx3 = jnp.concatenate([fidx, jnp.full((pad,), N, jnp.int32)]).reshape(NW, KPW, CHUNK)
    efp = jnp.concatenate([ef_init, jnp.zeros((pad, EF), jnp.float32)], axis=0)
    zagg = jnp.zeros((AGG_PT, 2 * H), jnp.float32)
    zefc = jnp.zeros((EFC_PT, EF), jnp.float32)

    w_sr = p['W_msg_r'][:NF].reshape(NF, NET, H).transpose(1, 0, 2)
    w_sm = p['W_msg_m'][:NF].reshape(NF, NET, H).transpose(1, 0, 2)
    w_src = jnp.concatenate([w_sr, w_sm], axis=2)
    w_dr = p['W_msg_r'][NF:2 * NF].reshape(NF, NET, H).transpose(1, 0, 2)
    w_dm = p['W_msg_m'][NF:2 * NF].reshape(NF, NET, H).transpose(1, 0, 2)
    w_dst = jnp.concatenate([w_dr, w_dm], axis=2)
    bias = jnp.concatenate([p['b_msg_r'], p['b_msg_m']], axis=1)[:, None, :]
    t_s, t_d = _build_tables(nf_init, w_src, w_dst, bias)
    table_s = t_s.reshape(NET * NP, 2 * H)
    table_d = t_d.reshape(NET * NP, 2 * H)

    aggout, efcout = _sc_call(table_s, table_d, gidx3, dst3, fidx3, efp,
                              zagg, zefc)

    agg_t = aggout[:, :N, :]
    efc_t = efcout.reshape(2, NET, NP, EF)[:, :, :N, :]
    lc = _post_call(
        agg_t, efc_t, nf_init,
        node_type.reshape(N, 1), rewards.reshape(1, T),
        p['W_msg_r'][2 * NF:], p['W_msg_m'][2 * NF:],
        p['W_node_r'], p['W_node_m'], p['b_node_r'], p['b_node_m'],
        p['W_ih'][:H], p['W_ih'][H:H + 1], p['b_ih'].reshape(1, 3 * RH),
        p['W_hh'], p['b_hh'].reshape(1, 3 * RH),
        p['w_act'], p['b_act'].reshape(1, 1),
        p['W_c1'], p['b_c1'].reshape(1, 32), p['W_c2'], p['b_c2'].reshape(1, 1),
    )
    return _final_call(lc).reshape(N + 1)


# speedup vs baseline: 7.3840x; 1.1536x over previous
import jax
import jax.numpy as jnp
from jax import lax
from jax.experimental import pallas as pl
from jax.experimental.pallas import tpu as pltpu
from jax.experimental.pallas import tpu_sc as plsc

N = 10000
E = 320000
NF = 128
EF = 16
H = 32
RH = 32
T = 5
NET = 4
NNT = 2

NW = 32
CHUNK = 128
KPW = 79
EP = NW * KPW * CHUNK
NP = 10016
AGG_PT = 632
EFC_PT = 2504
AGG_R = AGG_PT * 16
EFC_R = EFC_PT * 16


def _table_body(nf_ref, ws_ref, wd_ref, b_ref, ts_ref, td_ref):
    nf = nf_ref[...]
    ts_ref[0, 0:N, :] = jnp.dot(nf, ws_ref[0], preferred_element_type=jnp.float32)
    td_ref[0, 0:N, :] = jnp.dot(nf, wd_ref[0],
                                preferred_element_type=jnp.float32) + b_ref[0]


def _build_tables(nf, w_src, w_dst, bias):
    return pl.pallas_call(
        _table_body,
        grid=(NET,),
        in_specs=[
            pl.BlockSpec((N, NF), lambda t: (0, 0)),
            pl.BlockSpec((1, NF, 2 * H), lambda t: (t, 0, 0)),
            pl.BlockSpec((1, NF, 2 * H), lambda t: (t, 0, 0)),
            pl.BlockSpec((1, 1, 2 * H), lambda t: (t, 0, 0)),
        ],
        out_specs=[
            pl.BlockSpec((1, NP, 2 * H), lambda t: (t, 0, 0)),
            pl.BlockSpec((1, NP, 2 * H), lambda t: (t, 0, 0)),
        ],
        out_shape=[
            jax.ShapeDtypeStruct((NET, NP, 2 * H), jnp.float32),
            jax.ShapeDtypeStruct((NET, NP, 2 * H), jnp.float32),
        ],
    )(nf, w_src, w_dst, bias)


def _sc_body(table_s, table_d, idx3, efp, zagg, zefc,
             aggout, efcout,
             agg_s, efc_s,
             idxv0, idxv1, rows0, rows1, rowsd0, rowsd1, efv0, efv1,
             s_idx0, s_idx1, s_ef0, s_ef1, s_g10, s_g11, s_g20, s_g21,
             s_a10, s_a11, s_a20, s_a21, s_e0, s_e1):
    idxv = (idxv0, idxv1)
    rows = (rows0, rows1)
    rowsd = (rowsd0, rowsd1)
    efv = (efv0, efv1)
    s_idx = (s_idx0, s_idx1)
    s_ef = (s_ef0, s_ef1)
    s_g1 = (s_g10, s_g11)
    s_g2 = (s_g20, s_g21)
    s_a1 = (s_a10, s_a11)
    s_a2 = (s_a20, s_a21)
    s_e = (s_e0, s_e1)

    cid = lax.axis_index("c")
    sid = lax.axis_index("s")
    wid = cid * 16 + sid
    pltpu.sync_copy(zagg, agg_s.at[pl.ds(sid * AGG_PT, AGG_PT)])
    pltpu.sync_copy(zefc, efc_s.at[pl.ds(sid * EFC_PT, EFC_PT)])
    plsc.subcore_barrier()

    def fire(j, b):
        cr = wid * KPW + j
        pltpu.async_copy(idx3.at[cr], idxv[b], s_idx[b])
        pltpu.async_copy(efp.at[pl.ds(cr * CHUNK, CHUNK)], efv[b], s_ef[b])
        pltpu.make_async_copy(idx3.at[cr], idxv[b], s_idx[b]).wait()
        pltpu.async_copy(table_s.at[idxv[b].at[0]], rows[b], s_g1[b])
        pltpu.async_copy(table_d.at[idxv[b].at[2]], rowsd[b], s_g2[b])

    def use(j, b):
        cr = wid * KPW + j
        pltpu.make_async_copy(efp.at[pl.ds(cr * CHUNK, CHUNK)], efv[b], s_ef[b]).wait()
        pltpu.make_async_copy(table_s.at[idxv[b].at[0]], rows[b], s_g1[b]).wait()
        pltpu.make_async_copy(table_d.at[idxv[b].at[2]], rowsd[b], s_g2[b]).wait()
        pltpu.async_copy(rows[b], agg_s.at[idxv[b].at[1]], s_a1[b], add=True)
        pltpu.async_copy(rowsd[b], agg_s.at[idxv[b].at[1]], s_a2[b], add=True)
        pltpu.async_copy(efv[b], efc_s.at[idxv[b].at[2]], s_e[b], add=True)

    def drain(b):
        pltpu.make_async_copy(rows[b], agg_s.at[idxv[b].at[1]], s_a1[b]).wait()
        pltpu.make_async_copy(rowsd[b], agg_s.at[idxv[b].at[1]], s_a2[b]).wait()
        pltpu.make_async_copy(efv[b], efc_s.at[idxv[b].at[2]], s_e[b]).wait()

    def outer(o, carry):
        for b in (0, 1):
            j = 2 * o + b

            @pl.when(j >= 2)
            def _():
                drain(b)

            @pl.when(j < KPW)
            def _():
                fire(j, b)

            @pl.when(jnp.logical_and(j >= 1, j <= KPW))
            def _():
                use(j - 1, 1 - b)
        return carry

    lax.fori_loop(0, (KPW + 1) // 2, outer, 0)
    drain((KPW - 1) % 2)
    plsc.subcore_barrier()
    pltpu.sync_copy(agg_s.at[pl.ds(sid * AGG_PT, AGG_PT)],
                    aggout.at[cid, pl.ds(sid * AGG_PT, AGG_PT)])
    pltpu.sync_copy(efc_s.at[pl.ds(sid * EFC_PT, EFC_PT)],
                    efcout.at[cid, pl.ds(sid * EFC_PT, EFC_PT)])


_SC_MESH = plsc.VectorSubcoreMesh(core_axis_name="c", subcore_axis_name="s",
                                  num_cores=2, num_subcores=16)

_sc_call = pl.kernel(
    _sc_body,
    out_type=(
        jax.ShapeDtypeStruct((2, AGG_R, 2 * H), jnp.float32),
        jax.ShapeDtypeStruct((2, EFC_R, EF), jnp.float32),
    ),
    mesh=_SC_MESH,
    compiler_params=pltpu.CompilerParams(use_tc_tiling_on_sc=False),
    scratch_types=[
        pltpu.VMEM_SHARED((AGG_R, 2 * H), jnp.float32),
        pltpu.VMEM_SHARED((EFC_R, EF), jnp.float32),
        pltpu.VMEM((3, CHUNK), jnp.int32),
        pltpu.VMEM((3, CHUNK), jnp.int32),
        pltpu.VMEM((CHUNK, 2 * H), jnp.float32),
        pltpu.VMEM((CHUNK, 2 * H), jnp.float32),
        pltpu.VMEM((CHUNK, 2 * H), jnp.float32),
        pltpu.VMEM((CHUNK, 2 * H), jnp.float32),
        pltpu.VMEM((CHUNK, EF), jnp.float32),
        pltpu.VMEM((CHUNK, EF), jnp.float32),
    ] + [pltpu.SemaphoreType.DMA] * 14,
)


BLK = 2000


def _post_body(agg2, efc2, nf_ref, nt, rew,
               We_r, We_m,
               Wn_r, Wn_m, bn_r, bn_m,
               Wih, wr, bih, Whh, bhh, wact, bact, Wc1, bc1, Wc2, bc2,
               out):
    nf = nf_ref[...]
    agg = agg2[0] + agg2[1]
    aggr = agg[:, :H]
    aggm = agg[:, H:]
    for t in range(NET):
        Ft = efc2[0, t] + efc2[1, t]
        aggr = aggr + jnp.dot(Ft, We_r[:, t * H:(t + 1) * H],
                              preferred_element_type=jnp.float32)
        aggm = aggm + jnp.dot(Ft, We_m[:, t * H:(t + 1) * H],
                              preferred_element_type=jnp.float32)

    ntv = nt[...]

    def node_update(aggx, Wn, bn):
        hcat = jnp.concatenate([aggx, nf], axis=1)
        ha = jnp.dot(hcat, Wn[...], preferred_element_type=jnp.float32)
        sel = jnp.where(ntv == 0, ha[:, :H] + bn[0:1, :], ha[:, H:] + bn[1:2, :])
        return jnp.tanh(sel)

    nfr = node_update(aggr, Wn_r, bn_r)
    nfm = node_update(aggm, Wn_m, bn_m)

    gx = jnp.dot(nfr, Wih[...], preferred_element_type=jnp.float32) + bih[...]
    h = jnp.zeros((BLK, RH), jnp.float32)
    for t in range(T):
        gi = gx + rew[0:1, t:t + 1] * wr[...]
        gh = jnp.dot(h, Whh[...], preferred_element_type=jnp.float32) + bhh[...]
        r = jax.nn.sigmoid(gi[:, :RH] + gh[:, :RH])
        z = jax.nn.sigmoid(gi[:, RH:2 * RH] + gh[:, RH:2 * RH])
        n = jnp.tanh(gi[:, 2 * RH:] + r * gh[:, 2 * RH:])
        h = (1.0 - z) * n + z * h

    sim2a = jnp.concatenate([h, nfm], axis=1)
    logits = jnp.dot(sim2a, wact[...], preferred_element_type=jnp.float32) + bact[0:1, :]
    crit = jnp.dot(jnp.maximum(jnp.dot(sim2a, Wc1[...], preferred_element_type=jnp.float32)
                               + bc1[...], 0.0),
                   Wc2[...], preferred_element_type=jnp.float32) + bc2[0:1, :]
    out[...] = jnp.concatenate([logits, crit], axis=1)


def _post_call(agg, efc, nf, nt, rew, *weights):
    return pl.pallas_call(
        _post_body,
        grid=(N // BLK,),
        in_specs=[
            pl.BlockSpec((2, BLK, 2 * H), lambda i: (0, i, 0)),
            pl.BlockSpec((2, NET, BLK, EF), lambda i: (0, 0, i, 0)),
            pl.BlockSpec((BLK, NF), lambda i: (i, 0)),
            pl.BlockSpec((BLK, 1), lambda i: (i, 0)),
            pl.BlockSpec((1, T), lambda i: (0, 0)),
        ] + [pl.BlockSpec(w.shape, lambda i, _r=len(w.shape): (0,) * _r)
             for w in weights],
        out_specs=pl.BlockSpec((BLK, 2), lambda i: (i, 0)),
        out_shape=jax.ShapeDtypeStruct((N, 2), jnp.float32),
    )(agg, efc, nf, nt, rew, *weights)


def _final_body(lc_ref, out_ref):
    logits = lc_ref[:, 0:1]
    crit = lc_ref[:, 1:2]
    m = jnp.max(logits, keepdims=True)
    ex = jnp.exp(logits - m)
    probs = ex / jnp.sum(ex, keepdims=True)
    val = jnp.sum(crit, keepdims=True) * (1.0 / N)
    out_ref[...] = jnp.concatenate([probs, val], axis=0)


def _final_call(lc):
    return pl.pallas_call(
        _final_body,
        out_shape=jax.ShapeDtypeStruct((N + 1, 1), jnp.float32),
    )(lc)


def kernel(nf_init, ef_init, rewards, params, edge_index, edge_type, node_type):
    p = params
    src = edge_index[0]
    dst = edge_index[1]
    et = edge_type
    gidx = et * NP + src
    fidx = et * NP + dst
    pad = EP - E
    g2d = jnp.concatenate([gidx, jnp.full((pad,), N, jnp.int32)]).reshape(NW * KPW, CHUNK)
    d2d = jnp.concatenate([dst, jnp.full((pad,), N, jnp.int32)]).reshape(NW * KPW, CHUNK)
    f2d = jnp.concatenate([fidx, jnp.full((pad,), N, jnp.int32)]).reshape(NW * KPW, CHUNK)
    idx3 = jnp.stack([g2d, d2d, f2d], axis=1)
    efp = jnp.concatenate([ef_init, jnp.zeros((pad, EF), jnp.float32)], axis=0)
    zagg = jnp.zeros((AGG_PT, 2 * H), jnp.float32)
    zefc = jnp.zeros((EFC_PT, EF), jnp.float32)

    w_sr = p['W_msg_r'][:NF].reshape(NF, NET, H).transpose(1, 0, 2)
    w_sm = p['W_msg_m'][:NF].reshape(NF, NET, H).transpose(1, 0, 2)
    w_src = jnp.concatenate([w_sr, w_sm], axis=2)
    w_dr = p['W_msg_r'][NF:2 * NF].reshape(NF, NET, H).transpose(1, 0, 2)
    w_dm = p['W_msg_m'][NF:2 * NF].reshape(NF, NET, H).transpose(1, 0, 2)
    w_dst = jnp.concatenate([w_dr, w_dm], axis=2)
    bias = jnp.concatenate([p['b_msg_r'], p['b_msg_m']], axis=1)[:, None, :]
    t_s, t_d = _build_tables(nf_init, w_src, w_dst, bias)
    table_s = t_s.reshape(NET * NP, 2 * H)
    table_d = t_d.reshape(NET * NP, 2 * H)

    aggout, efcout = _sc_call(table_s, table_d, idx3, efp, zagg, zefc)

    agg_t = aggout[:, :N, :]
    efc_t = efcout.reshape(2, NET, NP, EF)[:, :, :N, :]
    lc = _post_call(
        agg_t, efc_t, nf_init,
        node_type.reshape(N, 1), rewards.reshape(1, T),
        p['W_msg_r'][2 * NF:], p['W_msg_m'][2 * NF:],
        p['W_node_r'], p['W_node_m'], p['b_node_r'], p['b_node_m'],
        p['W_ih'][:H], p['W_ih'][H:H + 1], p['b_ih'].reshape(1, 3 * RH),
        p['W_hh'], p['b_hh'].reshape(1, 3 * RH),
        p['w_act'], p['b_act'].reshape(1, 1),
        p['W_c1'], p['b_c1'].reshape(1, 32), p['W_c2'], p['b_c2'].reshape(1, 1),
    )
    return _final_call(lc).reshape(N + 1)


# speedup vs baseline: 11.2564x; 1.5244x over previous
import jax
import jax.numpy as jnp
from jax import lax
from jax.experimental import pallas as pl
from jax.experimental.pallas import tpu as pltpu
from jax.experimental.pallas import tpu_sc as plsc

N = 10000
E = 320000
NF = 128
EF = 16
H = 32
RH = 32
T = 5
NET = 4
NNT = 2

NW = 32
CHUNK = 128
NCH = E // CHUNK
KPW = (NCH + NW - 1) // NW
NP = 10016
AGG_PT = 632
EFC_PT = 2504
AGG_R = AGG_PT * 16
EFC_R = EFC_PT * 16


def _table_body(nf_ref, ws_ref, wd_ref, b_ref, ts_ref, td_ref):
    nf = nf_ref[...]
    ts_ref[0:N, :] = jnp.dot(nf, ws_ref[0], preferred_element_type=jnp.float32)
    td_ref[0:N, :] = jnp.dot(nf, wd_ref[0],
                             preferred_element_type=jnp.float32) + b_ref[0]


def _build_tables(nf, w_src, w_dst, bias):
    return pl.pallas_call(
        _table_body,
        grid=(NET,),
        in_specs=[
            pl.BlockSpec((N, NF), lambda t: (0, 0)),
            pl.BlockSpec((1, NF, 2 * H), lambda t: (t, 0, 0)),
            pl.BlockSpec((1, NF, 2 * H), lambda t: (t, 0, 0)),
            pl.BlockSpec((1, 1, 2 * H), lambda t: (t, 0, 0)),
        ],
        out_specs=[
            pl.BlockSpec((NP, 2 * H), lambda t: (t, 0)),
            pl.BlockSpec((NP, 2 * H), lambda t: (t, 0)),
        ],
        out_shape=[
            jax.ShapeDtypeStruct((NET * NP, 2 * H), jnp.float32),
            jax.ShapeDtypeStruct((NET * NP, 2 * H), jnp.float32),
        ],
    )(nf, w_src, w_dst, bias)


def _sc_body(table_s, table_d, g2d, d2d, f2d, efp, zagg, zefc,
             aggout, efcout,
             agg_s, efc_s,
             idxg0, idxg1, idxd0, idxd1, idxf0, idxf1,
             rows0, rows1, rowsd0, rowsd1, efv0, efv1,
             s_idx0, s_idx1, s_ef0, s_ef1, s_g10, s_g11, s_g20, s_g21,
             s_a10, s_a11, s_a20, s_a21, s_e0, s_e1):
    idxg = (idxg0, idxg1)
    idxd = (idxd0, idxd1)
    idxf = (idxf0, idxf1)
    rows = (rows0, rows1)
    rowsd = (rowsd0, rowsd1)
    efv = (efv0, efv1)
    s_idx = (s_idx0, s_idx1)
    s_ef = (s_ef0, s_ef1)
    s_g1 = (s_g10, s_g11)
    s_g2 = (s_g20, s_g21)
    s_a1 = (s_a10, s_a11)
    s_a2 = (s_a20, s_a21)
    s_e = (s_e0, s_e1)

    cid = lax.axis_index("c")
    sid = lax.axis_index("s")
    wid = cid * 16 + sid
    pltpu.sync_copy(zagg, agg_s.at[pl.ds(sid * AGG_PT, AGG_PT)])
    pltpu.sync_copy(zefc, efc_s.at[pl.ds(sid * EFC_PT, EFC_PT)])
    plsc.subcore_barrier()

    def cr_of(j):
        return j * NW + wid

    def fire(j, b):
        cr = cr_of(j)
        pltpu.async_copy(g2d.at[cr], idxg[b], s_idx[b])
        pltpu.async_copy(d2d.at[cr], idxd[b], s_idx[b])
        pltpu.async_copy(f2d.at[cr], idxf[b], s_idx[b])
        pltpu.async_copy(efp.at[pl.ds(cr * CHUNK, CHUNK)], efv[b], s_ef[b])
        pltpu.make_async_copy(g2d.at[cr], idxg[b], s_idx[b]).wait()
        pltpu.make_async_copy(d2d.at[cr], idxd[b], s_idx[b]).wait()
        pltpu.make_async_copy(f2d.at[cr], idxf[b], s_idx[b]).wait()
        pltpu.async_copy(table_s.at[idxg[b]], rows[b], s_g1[b])
        pltpu.async_copy(table_d.at[idxf[b]], rowsd[b], s_g2[b])

    def use(j, b):
        cr = cr_of(j)
        pltpu.make_async_copy(efp.at[pl.ds(cr * CHUNK, CHUNK)], efv[b], s_ef[b]).wait()
        pltpu.make_async_copy(table_s.at[idxg[b]], rows[b], s_g1[b]).wait()
        pltpu.make_async_copy(table_d.at[idxf[b]], rowsd[b], s_g2[b]).wait()
        pltpu.async_copy(rows[b], agg_s.at[idxd[b]], s_a1[b], add=True)
        pltpu.async_copy(rowsd[b], agg_s.at[idxd[b]], s_a2[b], add=True)
        pltpu.async_copy(efv[b], efc_s.at[idxf[b]], s_e[b], add=True)

    def drain(b):
        pltpu.make_async_copy(rows[b], agg_s.at[idxd[b]], s_a1[b]).wait()
        pltpu.make_async_copy(rowsd[b], agg_s.at[idxd[b]], s_a2[b]).wait()
        pltpu.make_async_copy(efv[b], efc_s.at[idxf[b]], s_e[b]).wait()

    def outer(o, carry):
        for b in (0, 1):
            j = 2 * o + b

            @pl.when(jnp.logical_and(j >= 2, cr_of(j - 2) < NCH))
            def _():
                drain(b)

            @pl.when(cr_of(j) < NCH)
            def _():
                fire(j, b)

            @pl.when(jnp.logical_and(j >= 1, cr_of(j - 1) < NCH))
            def _():
                use(j - 1, 1 - b)
        return carry

    lax.fori_loop(0, (KPW + 1) // 2, outer, 0)

    @pl.when(cr_of(KPW - 1) < NCH)
    def _():
        drain((KPW - 1) % 2)

    plsc.subcore_barrier()
    pltpu.sync_copy(agg_s.at[pl.ds(sid * AGG_PT, AGG_PT)],
                    aggout.at[cid, pl.ds(sid * AGG_PT, AGG_PT)])
    pltpu.sync_copy(efc_s.at[pl.ds(sid * EFC_PT, EFC_PT)],
                    efcout.at[cid, pl.ds(sid * EFC_PT, EFC_PT)])


_SC_MESH = plsc.VectorSubcoreMesh(core_axis_name="c", subcore_axis_name="s",
                                  num_cores=2, num_subcores=16)

_sc_call = pl.kernel(
    _sc_body,
    out_type=(
        jax.ShapeDtypeStruct((2, AGG_R, 2 * H), jnp.float32),
        jax.ShapeDtypeStruct((2, EFC_R, EF), jnp.float32),
    ),
    mesh=_SC_MESH,
    compiler_params=pltpu.CompilerParams(use_tc_tiling_on_sc=False),
    scratch_types=[
        pltpu.VMEM_SHARED((AGG_R, 2 * H), jnp.float32),
        pltpu.VMEM_SHARED((EFC_R, EF), jnp.float32),
        pltpu.VMEM((CHUNK,), jnp.int32),
        pltpu.VMEM((CHUNK,), jnp.int32),
        pltpu.VMEM((CHUNK,), jnp.int32),
        pltpu.VMEM((CHUNK,), jnp.int32),
        pltpu.VMEM((CHUNK,), jnp.int32),
        pltpu.VMEM((CHUNK,), jnp.int32),
        pltpu.VMEM((CHUNK, 2 * H), jnp.float32),
        pltpu.VMEM((CHUNK, 2 * H), jnp.float32),
        pltpu.VMEM((CHUNK, 2 * H), jnp.float32),
        pltpu.VMEM((CHUNK, 2 * H), jnp.float32),
        pltpu.VMEM((CHUNK, EF), jnp.float32),
        pltpu.VMEM((CHUNK, EF), jnp.float32),
    ] + [pltpu.SemaphoreType.DMA] * 14,
)


BLK = 2000


def _post_body(agg2, efc2, nf_ref, nt, rew,
               We_r, We_m,
               Wn_r, Wn_m, bn_r, bn_m,
               Wih, wr, bih, Whh, bhh, wact, bact, Wc1, bc1, Wc2, bc2,
               out):
    nf = nf_ref[...]
    agg = agg2[0] + agg2[1]
    aggr = agg[:, :H]
    aggm = agg[:, H:]
    for t in range(NET):
        Ft = efc2[0, t] + efc2[1, t]
        aggr = aggr + jnp.dot(Ft, We_r[:, t * H:(t + 1) * H],
                              preferred_element_type=jnp.float32)
        aggm = aggm + jnp.dot(Ft, We_m[:, t * H:(t + 1) * H],
                              preferred_element_type=jnp.float32)

    ntv = nt[...]

    def node_update(aggx, Wn, bn):
        hcat = jnp.concatenate([aggx, nf], axis=1)
        ha = jnp.dot(hcat, Wn[...], preferred_element_type=jnp.float32)
        sel = jnp.where(ntv == 0, ha[:, :H] + bn[0:1, :], ha[:, H:] + bn[1:2, :])
        return jnp.tanh(sel)

    nfr = node_update(aggr, Wn_r, bn_r)
    nfm = node_update(aggm, Wn_m, bn_m)

    gx = jnp.dot(nfr, Wih[...], preferred_element_type=jnp.float32) + bih[...]
    h = jnp.zeros((BLK, RH), jnp.float32)
    for t in range(T):
        gi = gx + rew[0:1, t:t + 1] * wr[...]
        gh = jnp.dot(h, Whh[...], preferred_element_type=jnp.float32) + bhh[...]
        r = jax.nn.sigmoid(gi[:, :RH] + gh[:, :RH])
        z = jax.nn.sigmoid(gi[:, RH:2 * RH] + gh[:, RH:2 * RH])
        n = jnp.tanh(gi[:, 2 * RH:] + r * gh[:, 2 * RH:])
        h = (1.0 - z) * n + z * h

    sim2a = jnp.concatenate([h, nfm], axis=1)
    logits = jnp.dot(sim2a, wact[...], preferred_element_type=jnp.float32) + bact[0:1, :]
    crit = jnp.dot(jnp.maximum(jnp.dot(sim2a, Wc1[...], preferred_element_type=jnp.float32)
                               + bc1[...], 0.0),
                   Wc2[...], preferred_element_type=jnp.float32) + bc2[0:1, :]
    out[...] = jnp.concatenate([logits, crit], axis=1)


def _post_call(agg, efc, nf, nt, rew, *weights):
    return pl.pallas_call(
        _post_body,
        grid=(N // BLK,),
        in_specs=[
            pl.BlockSpec((2, BLK, 2 * H), lambda i: (0, i, 0)),
            pl.BlockSpec((2, NET, BLK, EF), lambda i: (0, 0, i, 0)),
            pl.BlockSpec((BLK, NF), lambda i: (i, 0)),
            pl.BlockSpec((BLK, 1), lambda i: (i, 0)),
            pl.BlockSpec((1, T), lambda i: (0, 0)),
        ] + [pl.BlockSpec(w.shape, lambda i, _r=len(w.shape): (0,) * _r)
             for w in weights],
        out_specs=pl.BlockSpec((BLK, 2), lambda i: (i, 0)),
        out_shape=jax.ShapeDtypeStruct((N, 2), jnp.float32),
    )(agg, efc, nf, nt, rew, *weights)


def _final_body(lc_ref, out_ref):
    logits = lc_ref[:, 0:1]
    crit = lc_ref[:, 1:2]
    m = jnp.max(logits, keepdims=True)
    ex = jnp.exp(logits - m)
    probs = ex / jnp.sum(ex, keepdims=True)
    val = jnp.sum(crit, keepdims=True) * (1.0 / N)
    out_ref[...] = jnp.concatenate([probs, val], axis=0)


def _final_call(lc):
    return pl.pallas_call(
        _final_body,
        out_shape=jax.ShapeDtypeStruct((N + 1, 1), jnp.float32),
    )(lc)


def kernel(nf_init, ef_init, rewards, params, edge_index, edge_type, node_type):
    p = params
    src = edge_index[0]
    dst = edge_index[1]
    et = edge_type
    g2d = (et * NP + src).reshape(NCH, CHUNK)
    d2d = dst.reshape(NCH, CHUNK)
    f2d = (et * NP + dst).reshape(NCH, CHUNK)
    zagg = jnp.zeros((AGG_PT, 2 * H), jnp.float32)
    zefc = jnp.zeros((EFC_PT, EF), jnp.float32)

    w_sr = p['W_msg_r'][:NF].reshape(NF, NET, H).transpose(1, 0, 2)
    w_sm = p['W_msg_m'][:NF].reshape(NF, NET, H).transpose(1, 0, 2)
    w_src = jnp.concatenate([w_sr, w_sm], axis=2)
    w_dr = p['W_msg_r'][NF:2 * NF].reshape(NF, NET, H).transpose(1, 0, 2)
    w_dm = p['W_msg_m'][NF:2 * NF].reshape(NF, NET, H).transpose(1, 0, 2)
    w_dst = jnp.concatenate([w_dr, w_dm], axis=2)
    bias = jnp.concatenate([p['b_msg_r'], p['b_msg_m']], axis=1)[:, None, :]
    table_s, table_d = _build_tables(nf_init, w_src, w_dst, bias)

    aggout, efcout = _sc_call(table_s, table_d, g2d, d2d, f2d, ef_init,
                              zagg, zefc)

    efc4 = efcout.reshape(2, NET, NP, EF)
    lc = _post_call(
        aggout, efc4, nf_init,
        node_type.reshape(N, 1), rewards.reshape(1, T),
        p['W_msg_r'][2 * NF:], p['W_msg_m'][2 * NF:],
        p['W_node_r'], p['W_node_m'], p['b_node_r'], p['b_node_m'],
        p['W_ih'][:H], p['W_ih'][H:H + 1], p['b_ih'].reshape(1, 3 * RH),
        p['W_hh'], p['b_hh'].reshape(1, 3 * RH),
        p['w_act'], p['b_act'].reshape(1, 1),
        p['W_c1'], p['b_c1'].reshape(1, 32), p['W_c2'], p['b_c2'].reshape(1, 1),
    )
    return _final_call(lc).reshape(N + 1)


# speedup vs baseline: 11.4565x; 1.0178x over previous
import jax
import jax.numpy as jnp
from jax import lax
from jax.experimental import pallas as pl
from jax.experimental.pallas import tpu as pltpu
from jax.experimental.pallas import tpu_sc as plsc

N = 10000
E = 320000
NF = 128
EF = 16
H = 32
RH = 32
T = 5
NET = 4
NNT = 2

NW = 32
CHUNK = 128
NCH = E // CHUNK
KPW = (NCH + NW - 1) // NW
NP = 10016
AGG_PT = 632
EFC_PT = 2504
AGG_R = AGG_PT * 16
EFC_R = EFC_PT * 16


NH = N // 2
NPH = NP // 2


def _table_body(nf_ref, ws_ref, wd_ref, b_ref, ts_ref, td_ref):
    nf = nf_ref[...]
    ts_ref[0:NH, :] = jnp.dot(nf, ws_ref[0], preferred_element_type=jnp.float32)
    td_ref[0:NH, :] = jnp.dot(nf, wd_ref[0],
                              preferred_element_type=jnp.float32) + b_ref[0]


def _build_tables(nf2, w_src2, w_dst2, bias2):
    return pl.pallas_call(
        _table_body,
        grid=(NET,),
        in_specs=[
            pl.BlockSpec((NH, 2 * NF), lambda t: (0, 0)),
            pl.BlockSpec((1, 2 * NF, 4 * H), lambda t: (t, 0, 0)),
            pl.BlockSpec((1, 2 * NF, 4 * H), lambda t: (t, 0, 0)),
            pl.BlockSpec((1, 1, 4 * H), lambda t: (t, 0, 0)),
        ],
        out_specs=[
            pl.BlockSpec((NPH, 4 * H), lambda t: (t, 0)),
            pl.BlockSpec((NPH, 4 * H), lambda t: (t, 0)),
        ],
        out_shape=[
            jax.ShapeDtypeStruct((NET * NPH, 4 * H), jnp.float32),
            jax.ShapeDtypeStruct((NET * NPH, 4 * H), jnp.float32),
        ],
    )(nf2, w_src2, w_dst2, bias2)


def _sc_body(table_s, table_d, g2d, d2d, f2d, eft, zagg, zefc,
             aggout, efcout,
             agg_s, efc_s,
             idxg0, idxg1, idxd0, idxd1, idxf0, idxf1,
             rows0, rows1, rowsd0, rowsd1, efs0, efs1, efv0, efv1,
             s_idx0, s_idx1, s_ef0, s_ef1, s_g10, s_g11, s_g20, s_g21,
             s_a10, s_a11, s_a20, s_a21, s_e0, s_e1):
    idxg = (idxg0, idxg1)
    idxd = (idxd0, idxd1)
    idxf = (idxf0, idxf1)
    rows = (rows0, rows1)
    rowsd = (rowsd0, rowsd1)
    efs = (efs0, efs1)
    efv = (efv0, efv1)
    s_idx = (s_idx0, s_idx1)
    s_ef = (s_ef0, s_ef1)
    s_g1 = (s_g10, s_g11)
    s_g2 = (s_g20, s_g21)
    s_a1 = (s_a10, s_a11)
    s_a2 = (s_a20, s_a21)
    s_e = (s_e0, s_e1)

    cid = lax.axis_index("c")
    sid = lax.axis_index("s")
    wid = cid * 16 + sid
    pltpu.sync_copy(zagg, agg_s.at[pl.ds(sid * AGG_PT, AGG_PT)])
    pltpu.sync_copy(zefc, efc_s.at[pl.ds(sid * EFC_PT, EFC_PT)])
    plsc.subcore_barrier()

    def cr_of(j):
        return j * NW + wid

    iota16 = lax.iota(jnp.int32, 16)

    def fire(j, b):
        cr = cr_of(j)
        pltpu.async_copy(g2d.at[cr], idxg[b], s_idx[b])
        pltpu.async_copy(d2d.at[cr], idxd[b], s_idx[b])
        pltpu.async_copy(f2d.at[cr], idxf[b], s_idx[b])
        pltpu.async_copy(eft.at[:, pl.ds(cr * CHUNK, CHUNK)], efs[b], s_ef[b])
        pltpu.make_async_copy(g2d.at[cr], idxg[b], s_idx[b]).wait()
        pltpu.make_async_copy(d2d.at[cr], idxd[b], s_idx[b]).wait()
        pltpu.make_async_copy(f2d.at[cr], idxf[b], s_idx[b]).wait()
        pltpu.async_copy(table_s.at[idxg[b]], rows[b], s_g1[b])
        pltpu.async_copy(table_d.at[idxf[b]], rowsd[b], s_g2[b])

    def use(j, b):
        cr = cr_of(j)
        pltpu.make_async_copy(eft.at[:, pl.ds(cr * CHUNK, CHUNK)], efs[b], s_ef[b]).wait()

        def transpose_col(c, carry):
            col = plsc.load_gather(efs[b], [iota16, jnp.full((16,), c, jnp.int32)])
            efv[b][c, :] = col
            return carry

        lax.fori_loop(0, CHUNK, transpose_col, 0)
        pltpu.make_async_copy(table_s.at[idxg[b]], rows[b], s_g1[b]).wait()
        pltpu.make_async_copy(table_d.at[idxf[b]], rowsd[b], s_g2[b]).wait()
        pltpu.async_copy(rows[b], agg_s.at[idxd[b]], s_a1[b], add=True)
        pltpu.async_copy(rowsd[b], agg_s.at[idxd[b]], s_a2[b], add=True)
        pltpu.async_copy(efv[b], efc_s.at[idxf[b]], s_e[b], add=True)

    def drain(b):
        pltpu.make_async_copy(rows[b], agg_s.at[idxd[b]], s_a1[b]).wait()
        pltpu.make_async_copy(rowsd[b], agg_s.at[idxd[b]], s_a2[b]).wait()
        pltpu.make_async_copy(efv[b], efc_s.at[idxf[b]], s_e[b]).wait()

    def outer(o, carry):
        for b in (0, 1):
            j = 2 * o + b

            @pl.when(jnp.logical_and(j >= 2, cr_of(j - 2) < NCH))
            def _():
                drain(b)

            @pl.when(cr_of(j) < NCH)
            def _():
                fire(j, b)

            @pl.when(jnp.logical_and(j >= 1, cr_of(j - 1) < NCH))
            def _():
                use(j - 1, 1 - b)
        return carry

    lax.fori_loop(0, (KPW + 1) // 2, outer, 0)

    @pl.when(cr_of(KPW - 1) < NCH)
    def _():
        drain((KPW - 1) % 2)

    plsc.subcore_barrier()
    pltpu.sync_copy(agg_s.at[pl.ds(sid * AGG_PT, AGG_PT)],
                    aggout.at[cid, pl.ds(sid * AGG_PT, AGG_PT)])
    pltpu.sync_copy(efc_s.at[pl.ds(sid * EFC_PT, EFC_PT)],
                    efcout.at[cid, pl.ds(sid * EFC_PT, EFC_PT)])


_SC_MESH = plsc.VectorSubcoreMesh(core_axis_name="c", subcore_axis_name="s",
                                  num_cores=2, num_subcores=16)

_sc_call = pl.kernel(
    _sc_body,
    out_type=(
        jax.ShapeDtypeStruct((2, AGG_R, 2 * H), jnp.float32),
        jax.ShapeDtypeStruct((2, EFC_R, EF), jnp.float32),
    ),
    mesh=_SC_MESH,
    compiler_params=pltpu.CompilerParams(use_tc_tiling_on_sc=False,
                                         needs_layout_passes=False),
    scratch_types=[
        pltpu.VMEM_SHARED((AGG_R, 2 * H), jnp.float32),
        pltpu.VMEM_SHARED((EFC_R, EF), jnp.float32),
        pltpu.VMEM((CHUNK,), jnp.int32),
        pltpu.VMEM((CHUNK,), jnp.int32),
        pltpu.VMEM((CHUNK,), jnp.int32),
        pltpu.VMEM((CHUNK,), jnp.int32),
        pltpu.VMEM((CHUNK,), jnp.int32),
        pltpu.VMEM((CHUNK,), jnp.int32),
        pltpu.VMEM((CHUNK, 2 * H), jnp.float32),
        pltpu.VMEM((CHUNK, 2 * H), jnp.float32),
        pltpu.VMEM((CHUNK, 2 * H), jnp.float32),
        pltpu.VMEM((CHUNK, 2 * H), jnp.float32),
        pltpu.VMEM((EF, CHUNK), jnp.float32),
        pltpu.VMEM((EF, CHUNK), jnp.float32),
        pltpu.VMEM((CHUNK, EF), jnp.float32),
        pltpu.VMEM((CHUNK, EF), jnp.float32),
    ] + [pltpu.SemaphoreType.DMA] * 14,
)


BLK = 2000


def _post_body(agg2, efc2, nf_ref, nt, rew,
               We_r, We_m,
               Wn_r, Wn_m, bn_r, bn_m,
               Wih, wr, bih, Whh, bhh, wact, bact, Wc1, bc1, Wc2, bc2,
               out):
    nf = nf_ref[...]
    agg = agg2[0] + agg2[1]
    aggr = agg[:, :H]
    aggm = agg[:, H:]
    for t in range(NET):
        Ft = efc2[0, t] + efc2[1, t]
        aggr = aggr + jnp.dot(Ft, We_r[:, t * H:(t + 1) * H],
                              preferred_element_type=jnp.float32)
        aggm = aggm + jnp.dot(Ft, We_m[:, t * H:(t + 1) * H],
                              preferred_element_type=jnp.float32)

    ntv = nt[...]

    def node_update(aggx, Wn, bn):
        hcat = jnp.concatenate([aggx, nf], axis=1)
        ha = jnp.dot(hcat, Wn[...], preferred_element_type=jnp.float32)
        sel = jnp.where(ntv == 0, ha[:, :H] + bn[0:1, :], ha[:, H:] + bn[1:2, :])
        return jnp.tanh(sel)

    nfr = node_update(aggr, Wn_r, bn_r)
    nfm = node_update(aggm, Wn_m, bn_m)

    gx = jnp.dot(nfr, Wih[...], preferred_element_type=jnp.float32) + bih[...]
    h = jnp.zeros((BLK, RH), jnp.float32)
    for t in range(T):
        gi = gx + rew[0:1, t:t + 1] * wr[...]
        gh = jnp.dot(h, Whh[...], preferred_element_type=jnp.float32) + bhh[...]
        r = jax.nn.sigmoid(gi[:, :RH] + gh[:, :RH])
        z = jax.nn.sigmoid(gi[:, RH:2 * RH] + gh[:, RH:2 * RH])
        n = jnp.tanh(gi[:, 2 * RH:] + r * gh[:, 2 * RH:])
        h = (1.0 - z) * n + z * h

    sim2a = jnp.concatenate([h, nfm], axis=1)
    logits = jnp.dot(sim2a, wact[...], preferred_element_type=jnp.float32) + bact[0:1, :]
    crit = jnp.dot(jnp.maximum(jnp.dot(sim2a, Wc1[...], preferred_element_type=jnp.float32)
                               + bc1[...], 0.0),
                   Wc2[...], preferred_element_type=jnp.float32) + bc2[0:1, :]
    out[...] = jnp.concatenate([logits, crit], axis=1)


def _post_call(agg, efc, nf, nt, rew, *weights):
    return pl.pallas_call(
        _post_body,
        grid=(N // BLK,),
        in_specs=[
            pl.BlockSpec((2, BLK, 2 * H), lambda i: (0, i, 0)),
            pl.BlockSpec((2, NET, BLK, EF), lambda i: (0, 0, i, 0)),
            pl.BlockSpec((BLK, NF), lambda i: (i, 0)),
            pl.BlockSpec((BLK, 1), lambda i: (i, 0)),
            pl.BlockSpec((1, T), lambda i: (0, 0)),
        ] + [pl.BlockSpec(w.shape, lambda i, _r=len(w.shape): (0,) * _r)
             for w in weights],
        out_specs=pl.BlockSpec((BLK, 2), lambda i: (i, 0)),
        out_shape=jax.ShapeDtypeStruct((N, 2), jnp.float32),
    )(agg, efc, nf, nt, rew, *weights)


def _final_body(lc_ref, out_ref):
    logits = lc_ref[:, 0:1]
    crit = lc_ref[:, 1:2]
    m = jnp.max(logits, keepdims=True)
    ex = jnp.exp(logits - m)
    probs = ex / jnp.sum(ex, keepdims=True)
    val = jnp.sum(crit, keepdims=True) * (1.0 / N)
    out_ref[...] = jnp.concatenate([probs, val], axis=0)


def _final_call(lc):
    return pl.pallas_call(
        _final_body,
        out_shape=jax.ShapeDtypeStruct((N + 1, 1), jnp.float32),
    )(lc)


def kernel(nf_init, ef_init, rewards, params, edge_index, edge_type, node_type):
    p = params
    src = edge_index[0]
    dst = edge_index[1]
    et = edge_type
    g2d = (et * NP + src).reshape(NCH, CHUNK)
    d2d = dst.reshape(NCH, CHUNK)
    f2d = (et * NP + dst).reshape(NCH, CHUNK)
    zagg = jnp.zeros((AGG_PT, 2 * H), jnp.float32)
    zefc = jnp.zeros((EFC_PT, EF), jnp.float32)

    w_sr = p['W_msg_r'][:NF].reshape(NF, NET, H).transpose(1, 0, 2)
    w_sm = p['W_msg_m'][:NF].reshape(NF, NET, H).transpose(1, 0, 2)
    w_src = jnp.concatenate([w_sr, w_sm], axis=2)
    w_dr = p['W_msg_r'][NF:2 * NF].reshape(NF, NET, H).transpose(1, 0, 2)
    w_dm = p['W_msg_m'][NF:2 * NF].reshape(NF, NET, H).transpose(1, 0, 2)
    w_dst = jnp.concatenate([w_dr, w_dm], axis=2)
    bias = jnp.concatenate([p['b_msg_r'], p['b_msg_m']], axis=1)

    def blockdiag(w):
        z = jnp.zeros((NET, NF, 2 * H), jnp.float32)
        top = jnp.concatenate([w, z], axis=2)
        bot = jnp.concatenate([z, w], axis=2)
        return jnp.concatenate([top, bot], axis=1)

    nf2 = nf_init.reshape(NH, 2 * NF)
    bias2 = jnp.concatenate([bias, bias], axis=1)[:, None, :]
    tsp, tdp = _build_tables(nf2, blockdiag(w_src), blockdiag(w_dst), bias2)
    table_s = tsp.reshape(NET * NP, 2 * H)
    table_d = tdp.reshape(NET * NP, 2 * H)

    aggout, efcout = _sc_call(table_s, table_d, g2d, d2d, f2d, ef_init.T,
                              zagg, zefc)

    efc4 = efcout.reshape(2, NET, NP, EF)
    lc = _post_call(
        aggout, efc4, nf_init,
        node_type.reshape(N, 1), rewards.reshape(1, T),
        p['W_msg_r'][2 * NF:], p['W_msg_m'][2 * NF:],
        p['W_node_r'], p['W_node_m'], p['b_node_r'], p['b_node_m'],
        p['W_ih'][:H], p['W_ih'][H:H + 1], p['b_ih'].reshape(1, 3 * RH),
        p['W_hh'], p['b_hh'].reshape(1, 3 * RH),
        p['w_act'], p['b_act'].reshape(1, 1),
        p['W_c1'], p['b_c1'].reshape(1, 32), p['W_c2'], p['b_c2'].reshape(1, 1),
    )
    return _final_call(lc).reshape(N + 1)


# speedup vs baseline: 14.3530x; 1.2528x over previous
import jax
import jax.numpy as jnp
from jax import lax
from jax.experimental import pallas as pl
from jax.experimental.pallas import tpu as pltpu
from jax.experimental.pallas import tpu_sc as plsc

N = 10000
E = 320000
NF = 128
EF = 16
H = 32
RH = 32
T = 5
NET = 4
NNT = 2

NW = 32
CHUNK = 128
NCH = E // CHUNK
KPW = (NCH + NW - 1) // NW
NP = 10016
AGG_PT = 632
EFC_PT = 2504
AGG_R = AGG_PT * 16
EFC_R = EFC_PT * 16


NH = N // 2
NPH = NP // 2


def _table_body(nf_ref, ws_ref, wd_ref, b_ref, ts_ref, td_ref):
    nf = nf_ref[...]
    ts_ref[0:NH, :] = jnp.dot(nf, ws_ref[0], preferred_element_type=jnp.float32)
    td_ref[0:NH, :] = jnp.dot(nf, wd_ref[0],
                              preferred_element_type=jnp.float32) + b_ref[0]


def _build_tables(nf2, w_src2, w_dst2, bias2):
    return pl.pallas_call(
        _table_body,
        grid=(NET,),
        in_specs=[
            pl.BlockSpec((NH, 2 * NF), lambda t: (0, 0)),
            pl.BlockSpec((1, 2 * NF, 4 * H), lambda t: (t, 0, 0)),
            pl.BlockSpec((1, 2 * NF, 4 * H), lambda t: (t, 0, 0)),
            pl.BlockSpec((1, 1, 4 * H), lambda t: (t, 0, 0)),
        ],
        out_specs=[
            pl.BlockSpec((NPH, 4 * H), lambda t: (t, 0)),
            pl.BlockSpec((NPH, 4 * H), lambda t: (t, 0)),
        ],
        out_shape=[
            jax.ShapeDtypeStruct((NET * NPH, 4 * H), jnp.float32),
            jax.ShapeDtypeStruct((NET * NPH, 4 * H), jnp.float32),
        ],
    )(nf2, w_src2, w_dst2, bias2)


def _sc_body(table_s, table_d, g2d, d2d, f2d, eft, zagg, zefc,
             aggout, efcout,
             agg_s, efc_s,
             idxg0, idxg1, idxd0, idxd1, idxf0, idxf1,
             rows0, rows1, rowsd0, rowsd1, efs0, efs1, efv0, efv1,
             s_idx0, s_idx1, s_ef0, s_ef1, s_g10, s_g11, s_g20, s_g21,
             s_a10, s_a11, s_a20, s_a21, s_e0, s_e1):
    idxg = (idxg0, idxg1)
    idxd = (idxd0, idxd1)
    idxf = (idxf0, idxf1)
    rows = (rows0, rows1)
    rowsd = (rowsd0, rowsd1)
    efs = (efs0, efs1)
    efv = (efv0, efv1)
    s_idx = (s_idx0, s_idx1)
    s_ef = (s_ef0, s_ef1)
    s_g1 = (s_g10, s_g11)
    s_g2 = (s_g20, s_g21)
    s_a1 = (s_a10, s_a11)
    s_a2 = (s_a20, s_a21)
    s_e = (s_e0, s_e1)

    cid = lax.axis_index("c")
    sid = lax.axis_index("s")
    wid = cid * 16 + sid
    pltpu.sync_copy(zagg, agg_s.at[pl.ds(sid * AGG_PT, AGG_PT)])
    pltpu.sync_copy(zefc, efc_s.at[pl.ds(sid * EFC_PT, EFC_PT)])
    plsc.subcore_barrier()

    iota16 = lax.iota(jnp.int32, 16)

    def cr_of(j):
        return j * NW + wid

    def fire(j, b):
        cr = cr_of(j)
        pltpu.async_copy(g2d.at[cr], idxg[b], s_idx[b])
        pltpu.async_copy(d2d.at[cr], idxd[b], s_idx[b])
        pltpu.async_copy(f2d.at[cr], idxf[b], s_idx[b])
        pltpu.async_copy(eft.at[:, pl.ds(cr * CHUNK, CHUNK)], efs[b], s_ef[b])
        pltpu.make_async_copy(g2d.at[cr], idxg[b], s_idx[b]).wait()
        pltpu.make_async_copy(d2d.at[cr], idxd[b], s_idx[b]).wait()
        pltpu.make_async_copy(f2d.at[cr], idxf[b], s_idx[b]).wait()
        pltpu.async_copy(table_s.at[idxg[b]], rows[b], s_g1[b])
        pltpu.async_copy(table_d.at[idxf[b]], rowsd[b], s_g2[b])

    def use(j, b):
        cr = cr_of(j)
        pltpu.make_async_copy(eft.at[:, pl.ds(cr * CHUNK, CHUNK)], efs[b], s_ef[b]).wait()
        for blk in range(CHUNK // 16):
            ci = blk * 16 + iota16
            for f in range(EF):
                plsc.store_scatter(efv[b], [ci, jnp.full((16,), f, jnp.int32)],
                                   efs[b][f, pl.ds(blk * 16, 16)])
        pltpu.make_async_copy(table_s.at[idxg[b]], rows[b], s_g1[b]).wait()
        pltpu.make_async_copy(table_d.at[idxf[b]], rowsd[b], s_g2[b]).wait()
        pltpu.async_copy(rows[b], agg_s.at[idxd[b]], s_a1[b], add=True)
        pltpu.async_copy(rowsd[b], agg_s.at[idxd[b]], s_a2[b], add=True)
        pltpu.async_copy(efv[b], efc_s.at[idxf[b]], s_e[b], add=True)

    def drain(b):
        pltpu.make_async_copy(rows[b], agg_s.at[idxd[b]], s_a1[b]).wait()
        pltpu.make_async_copy(rowsd[b], agg_s.at[idxd[b]], s_a2[b]).wait()
        pltpu.make_async_copy(efv[b], efc_s.at[idxf[b]], s_e[b]).wait()

    def outer(o, carry):
        for b in (0, 1):
            j = 2 * o + b

            @pl.when(jnp.logical_and(j >= 2, cr_of(j - 2) < NCH))
            def _():
                drain(b)

            @pl.when(cr_of(j) < NCH)
            def _():
                fire(j, b)

            @pl.when(jnp.logical_and(j >= 1, cr_of(j - 1) < NCH))
            def _():
                use(j - 1, 1 - b)
        return carry

    lax.fori_loop(0, (KPW + 1) // 2, outer, 0)

    @pl.when(cr_of(KPW - 1) < NCH)
    def _():
        drain((KPW - 1) % 2)

    plsc.subcore_barrier()
    pltpu.sync_copy(agg_s.at[pl.ds(sid * AGG_PT, AGG_PT)],
                    aggout.at[cid, pl.ds(sid * AGG_PT, AGG_PT)])
    pltpu.sync_copy(efc_s.at[pl.ds(sid * EFC_PT, EFC_PT)],
                    efcout.at[cid, pl.ds(sid * EFC_PT, EFC_PT)])


_SC_MESH = plsc.VectorSubcoreMesh(core_axis_name="c", subcore_axis_name="s",
                                  num_cores=2, num_subcores=16)

_sc_call = pl.kernel(
    _sc_body,
    out_type=(
        jax.ShapeDtypeStruct((2, AGG_R, 2 * H), jnp.float32),
        jax.ShapeDtypeStruct((2, EFC_R, EF), jnp.float32),
    ),
    mesh=_SC_MESH,
    compiler_params=pltpu.CompilerParams(use_tc_tiling_on_sc=False,
                                         needs_layout_passes=False),
    scratch_types=[
        pltpu.VMEM_SHARED((AGG_R, 2 * H), jnp.float32),
        pltpu.VMEM_SHARED((EFC_R, EF), jnp.float32),
        pltpu.VMEM((CHUNK,), jnp.int32),
        pltpu.VMEM((CHUNK,), jnp.int32),
        pltpu.VMEM((CHUNK,), jnp.int32),
        pltpu.VMEM((CHUNK,), jnp.int32),
        pltpu.VMEM((CHUNK,), jnp.int32),
        pltpu.VMEM((CHUNK,), jnp.int32),
        pltpu.VMEM((CHUNK, 2 * H), jnp.float32),
        pltpu.VMEM((CHUNK, 2 * H), jnp.float32),
        pltpu.VMEM((CHUNK, 2 * H), jnp.float32),
        pltpu.VMEM((CHUNK, 2 * H), jnp.float32),
        pltpu.VMEM((EF, CHUNK), jnp.float32),
        pltpu.VMEM((EF, CHUNK), jnp.float32),
        pltpu.VMEM((CHUNK, EF), jnp.float32),
        pltpu.VMEM((CHUNK, EF), jnp.float32),
    ] + [pltpu.SemaphoreType.DMA] * 14,
)


BLK = 2000


def _post_body(agg2, efc2, nf_ref, nt, rew,
               We_r, We_m,
               Wn_r, Wn_m, bn_r, bn_m,
               Wih, wr, bih, Whh, bhh, wact, bact, Wc1, bc1, Wc2, bc2,
               out):
    nf = nf_ref[...]
    agg = agg2[0] + agg2[1]
    aggr = agg[:, :H]
    aggm = agg[:, H:]
    for t in range(NET):
        Ft = efc2[0, t] + efc2[1, t]
        aggr = aggr + jnp.dot(Ft, We_r[:, t * H:(t + 1) * H],
                              preferred_element_type=jnp.float32)
        aggm = aggm + jnp.dot(Ft, We_m[:, t * H:(t + 1) * H],
                              preferred_element_type=jnp.float32)

    ntv = nt[...]

    def node_update(aggx, Wn, bn):
        hcat = jnp.concatenate([aggx, nf], axis=1)
        ha = jnp.dot(hcat, Wn[...], preferred_element_type=jnp.float32)
        sel = jnp.where(ntv == 0, ha[:, :H] + bn[0:1, :], ha[:, H:] + bn[1:2, :])
        return jnp.tanh(sel)

    nfr = node_update(aggr, Wn_r, bn_r)
    nfm = node_update(aggm, Wn_m, bn_m)

    gx = jnp.dot(nfr, Wih[...], preferred_element_type=jnp.float32) + bih[...]
    h = jnp.zeros((BLK, RH), jnp.float32)
    for t in range(T):
        gi = gx + rew[0:1, t:t + 1] * wr[...]
        gh = jnp.dot(h, Whh[...], preferred_element_type=jnp.float32) + bhh[...]
        r = jax.nn.sigmoid(gi[:, :RH] + gh[:, :RH])
        z = jax.nn.sigmoid(gi[:, RH:2 * RH] + gh[:, RH:2 * RH])
        n = jnp.tanh(gi[:, 2 * RH:] + r * gh[:, 2 * RH:])
        h = (1.0 - z) * n + z * h

    sim2a = jnp.concatenate([h, nfm], axis=1)
    logits = jnp.dot(sim2a, wact[...], preferred_element_type=jnp.float32) + bact[0:1, :]
    crit = jnp.dot(jnp.maximum(jnp.dot(sim2a, Wc1[...], preferred_element_type=jnp.float32)
                               + bc1[...], 0.0),
                   Wc2[...], preferred_element_type=jnp.float32) + bc2[0:1, :]
    out[...] = jnp.concatenate([logits, crit], axis=1)


def _post_call(agg, efc, nf, nt, rew, *weights):
    return pl.pallas_call(
        _post_body,
        grid=(N // BLK,),
        in_specs=[
            pl.BlockSpec((2, BLK, 2 * H), lambda i: (0, i, 0)),
            pl.BlockSpec((2, NET, BLK, EF), lambda i: (0, 0, i, 0)),
            pl.BlockSpec((BLK, NF), lambda i: (i, 0)),
            pl.BlockSpec((BLK, 1), lambda i: (i, 0)),
            pl.BlockSpec((1, T), lambda i: (0, 0)),
        ] + [pl.BlockSpec(w.shape, lambda i, _r=len(w.shape): (0,) * _r)
             for w in weights],
        out_specs=pl.BlockSpec((BLK, 2), lambda i: (i, 0)),
        out_shape=jax.ShapeDtypeStruct((N, 2), jnp.float32),
    )(agg, efc, nf, nt, rew, *weights)


def _final_body(lc_ref, out_ref):
    logits = lc_ref[:, 0:1]
    crit = lc_ref[:, 1:2]
    m = jnp.max(logits, keepdims=True)
    ex = jnp.exp(logits - m)
    probs = ex / jnp.sum(ex, keepdims=True)
    val = jnp.sum(crit, keepdims=True) * (1.0 / N)
    out_ref[...] = jnp.concatenate([probs, val], axis=0)


def _final_call(lc):
    return pl.pallas_call(
        _final_body,
        out_shape=jax.ShapeDtypeStruct((N + 1, 1), jnp.float32),
    )(lc)


def kernel(nf_init, ef_init, rewards, params, edge_index, edge_type, node_type):
    p = params
    src = edge_index[0]
    dst = edge_index[1]
    et = edge_type
    g2d = (et * NP + src).reshape(NCH, CHUNK)
    d2d = dst.reshape(NCH, CHUNK)
    f2d = (et * NP + dst).reshape(NCH, CHUNK)
    zagg = jnp.zeros((AGG_PT, 2 * H), jnp.float32)
    zefc = jnp.zeros((EFC_PT, EF), jnp.float32)

    w_sr = p['W_msg_r'][:NF].reshape(NF, NET, H).transpose(1, 0, 2)
    w_sm = p['W_msg_m'][:NF].reshape(NF, NET, H).transpose(1, 0, 2)
    w_src = jnp.concatenate([w_sr, w_sm], axis=2)
    w_dr = p['W_msg_r'][NF:2 * NF].reshape(NF, NET, H).transpose(1, 0, 2)
    w_dm = p['W_msg_m'][NF:2 * NF].reshape(NF, NET, H).transpose(1, 0, 2)
    w_dst = jnp.concatenate([w_dr, w_dm], axis=2)
    bias = jnp.concatenate([p['b_msg_r'], p['b_msg_m']], axis=1)

    def blockdiag(w):
        z = jnp.zeros((NET, NF, 2 * H), jnp.float32)
        top = jnp.concatenate([w, z], axis=2)
        bot = jnp.concatenate([z, w], axis=2)
        return jnp.concatenate([top, bot], axis=1)

    nf2 = nf_init.reshape(NH, 2 * NF)
    bias2 = jnp.concatenate([bias, bias], axis=1)[:, None, :]
    tsp, tdp = _build_tables(nf2, blockdiag(w_src), blockdiag(w_dst), bias2)
    table_s = tsp.reshape(NET * NP, 2 * H)
    table_d = tdp.reshape(NET * NP, 2 * H)

    aggout, efcout = _sc_call(table_s, table_d, g2d, d2d, f2d, ef_init.T,
                              zagg, zefc)

    efc4 = efcout.reshape(2, NET, NP, EF)
    lc = _post_call(
        aggout, efc4, nf_init,
        node_type.reshape(N, 1), rewards.reshape(1, T),
        p['W_msg_r'][2 * NF:], p['W_msg_m'][2 * NF:],
        p['W_node_r'], p['W_node_m'], p['b_node_r'], p['b_node_m'],
        p['W_ih'][:H], p['W_ih'][H:H + 1], p['b_ih'].reshape(1, 3 * RH),
        p['W_hh'], p['b_hh'].reshape(1, 3 * RH),
        p['w_act'], p['b_act'].reshape(1, 1),
        p['W_c1'], p['b_c1'].reshape(1, 32), p['W_c2'], p['b_c2'].reshape(1, 1),
    )
    return _final_call(lc).reshape(N + 1)
